# baseline (speedup 1.0000x reference)
import jax
import jax.numpy as jnp
from jax.experimental import pallas as pl
from jax.experimental.pallas import tpu as pltpu

_BM = 400


def _gcn_block_kernel(x_ref, adj_ref, w_ref, b_ref, out_ref, support_ref):
    @pl.when(pl.program_id(0) == 0)
    def _():
        support_ref[...] = jnp.dot(
            x_ref[...], w_ref[...], preferred_element_type=jnp.float32
        )

    out = jnp.dot(
        adj_ref[...], support_ref[...], preferred_element_type=jnp.float32
    )
    h = jnp.maximum(out + b_ref[...], 0.0)
    m = jnp.max(h, axis=1, keepdims=True)
    lse = m + jnp.log(jnp.sum(jnp.exp(h - m), axis=1, keepdims=True))
    out_ref[...] = h - lse


def kernel(x, adj, W, b):
    n, feat = x.shape
    nclass = W.shape[1]
    b2 = b.reshape(1, nclass)
    return pl.pallas_call(
        _gcn_block_kernel,
        grid=(n // _BM,),
        in_specs=[
            pl.BlockSpec((n, feat), lambda i: (0, 0)),
            pl.BlockSpec((_BM, n), lambda i: (i, 0)),
            pl.BlockSpec((feat, nclass), lambda i: (0, 0)),
            pl.BlockSpec((1, nclass), lambda i: (0, 0)),
        ],
        out_specs=pl.BlockSpec((_BM, nclass), lambda i: (i, 0)),
        out_shape=jax.ShapeDtypeStruct((n, nclass), jnp.float32),
        scratch_shapes=[pltpu.VMEM((n, nclass), jnp.float32)],
    )(x, adj, W, b2)
